# baseline (speedup 1.0000x reference)
import functools

import jax
import jax.numpy as jnp
from jax import lax
from jax.experimental import pallas as pl
from jax.experimental.pallas import tpu as pltpu
from jax.experimental.pallas import tpu_sc as plsc

_N, _E, _D = 10000, 320000, 128
_NP = 10240
_NC, _NS = 2, 16
_NW = _NC * _NS
_EPW = _E // _NW
_RPT = _NP // _NS

_IC = 96
_IR = 108
_DW = 8

_PC = 128
_PR = 80
_CB = 128
_NB = _RPT // _CB

_mesh = plsc.VectorSubcoreMesh(core_axis_name="c", subcore_axis_name="s")
_sc_params = pltpu.CompilerParams(use_tc_tiling_on_sc=False)


def _fill_const(buf, rows, cols, val):
    ncol = cols // 16

    def body(i, carry):
        r = i // ncol
        c = i % ncol
        buf[r, pl.ds(c * 16, 16)] = jnp.full((16,), val, jnp.float32)
        return carry

    lax.fori_loop(0, rows * ncol, body, 0)


@functools.partial(
    pl.kernel,
    out_type=jax.ShapeDtypeStruct((_NC, _NP, 16), jnp.float32),
    mesh=_mesh,
    scratch_types=[
        pltpu.VMEM((_IR, _IC), jnp.int32),
        pltpu.VMEM((_IC, 16), jnp.float32),
        pltpu.VMEM((_RPT, 16), jnp.float32),
        pltpu.VMEM_SHARED((_NP, 16), jnp.float32),
        pltpu.SemaphoreType.DMA,
    ],
    compiler_params=_sc_params,
)
def _deg_kernel(dst_hbm, deg_out, didx_v, ones_v, buf_v, deg_sp, sem):
    cid = lax.axis_index("c")
    sid = lax.axis_index("s")
    wid = sid * _NC + cid

    pltpu.sync_copy(dst_hbm.at[wid], didx_v)
    _fill_const(ones_v, _IC, 16, 1.0)
    _fill_const(buf_v, _RPT, 16, 0.0)
    pltpu.sync_copy(buf_v, deg_sp.at[pl.ds(sid * _RPT, _RPT)])
    plsc.subcore_barrier()

    def start(i):
        pltpu.async_copy(ones_v, deg_sp.at[didx_v.at[i]], sem, add=True)

    def wait():
        pltpu.make_async_copy(ones_v, deg_sp.at[didx_v.at[0]], sem).wait()

    for k in range(_DW):
        start(k)

    def body(i, carry):
        wait()
        start(i + _DW)
        return carry

    lax.fori_loop(0, _IR - _DW, body, 0)
    for k in range(_DW):
        wait()
    plsc.subcore_barrier()
    pltpu.sync_copy(deg_sp.at[pl.ds(sid * _RPT, _RPT)], buf_v)
    pltpu.sync_copy(buf_v, deg_out.at[cid, pl.ds(sid * _RPT, _RPT)])


_msg_scratch = (
    [pltpu.VMEM((_PR, _PC), jnp.int32)]
    + [pltpu.VMEM((_PC,), jnp.int32) for _ in range(6)]
    + [pltpu.VMEM((_PC, _D), jnp.bfloat16) for _ in range(3)]
    + [pltpu.SemaphoreType.DMA for _ in range(6)]
    + [pltpu.VMEM_SHARED((_NP, _D), jnp.bfloat16)]
)


@functools.partial(
    pl.kernel,
    out_type=jax.ShapeDtypeStruct((_NC, _NP, _D), jnp.bfloat16),
    mesh=_mesh,
    scratch_types=_msg_scratch,
    compiler_params=_sc_params,
)
def _msg_kernel(y_hbm, pidx_hbm, acc_out, *sc):
    pidx_v = sc[0]
    slots = tuple((sc[1 + r], sc[4 + r], sc[7 + r], sc[10 + r], sc[13 + r])
                  for r in range(3))
    acc_sp = sc[16]

    cid = lax.axis_index("c")
    sid = lax.axis_index("s")
    wid = sid * _NC + cid

    pltpu.sync_copy(pidx_hbm.at[wid], pidx_v)

    buf0 = slots[0][2]

    def fz(i, carry):
        r = i // (_D // 32)
        c = i % (_D // 32)
        buf0[r, pl.ds(c * 32, 32)] = jnp.zeros((32,), jnp.bfloat16)
        return carry

    lax.fori_loop(0, _CB * (_D // 32), fz, 0)
    for t in range(_NB):
        pltpu.async_copy(buf0.at[pl.ds(0, _CB)],
                         acc_sp.at[pl.ds(sid * _RPT + t * _CB, _CB)], slots[0][3])
    for t in range(_NB):
        pltpu.make_async_copy(buf0.at[pl.ds(0, _CB)],
                              acc_sp.at[pl.ds(sid * _RPT, _CB)], slots[0][3]).wait()
    plsc.subcore_barrier()

    def unpack(v, sl):
        for c in range(_PC // 16):
            u = pidx_v[v, pl.ds(c * 16, 16)]
            sl[0][pl.ds(c * 16, 16)] = jnp.bitwise_and(u, 0xFFFF)
            sl[1][pl.ds(c * 16, 16)] = lax.shift_right_logical(u, 16)

    def gath(sl):
        pltpu.async_copy(y_hbm.at[sl[0]], sl[2], sl[3])

    def wait_g(sl):
        pltpu.make_async_copy(y_hbm.at[sl[0]], sl[2], sl[3]).wait()

    def scat(sl):
        pltpu.async_copy(sl[2], acc_sp.at[sl[1]], sl[4], add=True)

    def wait_s(sl):
        pltpu.make_async_copy(sl[2], acc_sp.at[sl[1]], sl[4]).wait()

    def step(i, c_sl, n_sl, first=False):
        nxt = slots[n_sl]
        if not first:
            wait_s(nxt)
        unpack(i + 1, nxt)
        gath(nxt)
        cur = slots[c_sl]
        wait_g(cur)
        scat(cur)

    unpack(0, slots[0])
    gath(slots[0])
    step(0, 0, 1, first=True)
    step(1, 1, 2, first=True)

    def body(k, carry):
        for r in range(3):
            step(2 + 3 * k + r, (2 + r) % 3, r)
        return carry

    lax.fori_loop(0, 25, body, 0)

    step(77, 77 % 3, 78 % 3)
    step(78, 78 % 3, 79 % 3)
    last = slots[79 % 3]
    wait_g(last)
    scat(last)
    for r in range(3):
        wait_s(slots[r])

    plsc.subcore_barrier()
    for t in range(_NB):
        blk = pl.ds(sid * _RPT + t * _CB, _CB)
        pltpu.sync_copy(acc_sp.at[blk], buf0.at[pl.ds(0, _CB)])
        pltpu.sync_copy(buf0.at[pl.ds(0, _CB)], acc_out.at[cid, blk])


def _mm_body(x_ref, w_ref, xw_ref):
    xw_ref[...] = jnp.dot(x_ref[...], w_ref[...], preferred_element_type=jnp.float32)


_mm = pl.pallas_call(
    _mm_body,
    out_shape=jax.ShapeDtypeStruct((_N, _D), jnp.float32),
)


def _scale_body(deg_ref, xw_ref, y_ref, ybf_ref, dis_ref):
    deg = deg_ref[0][:_N, 0:1] + deg_ref[1][:_N, 0:1] + 1.0
    dis = lax.rsqrt(deg)
    y = xw_ref[...] * dis
    y_ref[...] = y
    ybf_ref[...] = y.astype(jnp.bfloat16)
    dis_ref[...] = dis


_scale = pl.pallas_call(
    _scale_body,
    out_shape=(
        jax.ShapeDtypeStruct((_N, _D), jnp.float32),
        jax.ShapeDtypeStruct((_N, _D), jnp.bfloat16),
        jax.ShapeDtypeStruct((_N, 1), jnp.float32),
    ),
)


def _out_body(acc_ref, y_ref, dis_ref, b_ref, out_ref):
    acc = acc_ref[0][:_N].astype(jnp.float32) + acc_ref[1][:_N].astype(jnp.float32)
    out_ref[...] = (acc + y_ref[...]) * dis_ref[...] + b_ref[...]


_outk = pl.pallas_call(
    _out_body,
    out_shape=jax.ShapeDtypeStruct((_N, _D), jnp.float32),
)


@jax.jit
def _run(x, edge_index, W, b):
    src = edge_index[0].reshape(_NW, _EPW)
    dst = edge_index[1].reshape(_NW, _EPW)
    dpad = _IR * _IC - _EPW
    dst_deg = jnp.pad(dst, ((0, 0), (0, dpad)), constant_values=_N).reshape(_NW, _IR, _IC)
    packed = jnp.bitwise_or(src, jnp.left_shift(dst, 16))
    ppad = _PR * _PC - _EPW
    packed = jnp.pad(packed, ((0, 0), (0, ppad)),
                     constant_values=_N << 16).reshape(_NW, _PR, _PC)
    degp = _deg_kernel(dst_deg)
    xw = _mm(x, W)
    y, ybf, dis = _scale(degp, xw)
    accp = _msg_kernel(ybf, packed)
    return _outk(accp, y, dis, b.reshape(1, _D))


def kernel(x, edge_index, W, b):
    return _run(x, edge_index, W, b)


# speedup vs baseline: 1.8073x; 1.8073x over previous
import functools

import jax
import jax.numpy as jnp
from jax import lax
from jax.experimental import pallas as pl
from jax.experimental.pallas import tpu as pltpu
from jax.experimental.pallas import tpu_sc as plsc

_N, _E, _D = 10000, 320000, 128
_NP = 10240
_NC, _NS = 2, 16
_NW = _NC * _NS
_EPW = _E // _NW
_RPT = _NP // _NS

_IC = 80
_IR = 125
_DW = 12

_PC = 80
_PR = 125
_CB = 80
_NB = _RPT // _CB

_mesh = plsc.VectorSubcoreMesh(core_axis_name="c", subcore_axis_name="s")
_sc_params = pltpu.CompilerParams(use_tc_tiling_on_sc=False)


def _fill_const(buf, rows, cols, val):
    ncol = cols // 16

    def body(i, carry):
        r = i // ncol
        c = i % ncol
        buf[r, pl.ds(c * 16, 16)] = jnp.full((16,), val, jnp.float32)
        return carry

    lax.fori_loop(0, rows * ncol, body, 0)


@functools.partial(
    pl.kernel,
    out_type=jax.ShapeDtypeStruct((_NC, _NP, 16), jnp.float32),
    mesh=_mesh,
    scratch_types=[
        pltpu.VMEM((_IR, _IC), jnp.int32),
        pltpu.VMEM((_IC, 16), jnp.float32),
        pltpu.VMEM((_RPT, 16), jnp.float32),
        pltpu.VMEM_SHARED((_NP, 16), jnp.float32),
        pltpu.SemaphoreType.DMA,
    ],
    compiler_params=_sc_params,
)
def _deg_kernel(dst_hbm, deg_out, didx_v, ones_v, buf_v, deg_sp, sem):
    cid = lax.axis_index("c")
    sid = lax.axis_index("s")
    wid = sid * _NC + cid

    pltpu.sync_copy(dst_hbm.at[wid], didx_v)
    _fill_const(ones_v, _IC, 16, 1.0)
    _fill_const(buf_v, _RPT, 16, 0.0)
    pltpu.sync_copy(buf_v, deg_sp.at[pl.ds(sid * _RPT, _RPT)])
    plsc.subcore_barrier()

    def start(i):
        pltpu.async_copy(ones_v, deg_sp.at[didx_v.at[i]], sem, add=True)

    def wait():
        pltpu.make_async_copy(ones_v, deg_sp.at[didx_v.at[0]], sem).wait()

    for k in range(_DW):
        start(k)

    def body(i, carry):
        wait()
        start(i + _DW)
        return carry

    lax.fori_loop(0, _IR - _DW, body, 0)
    for k in range(_DW):
        wait()
    plsc.subcore_barrier()
    pltpu.sync_copy(deg_sp.at[pl.ds(sid * _RPT, _RPT)], buf_v)
    pltpu.sync_copy(buf_v, deg_out.at[cid, pl.ds(sid * _RPT, _RPT)])


_msg_scratch = (
    [pltpu.VMEM((_PR, _PC), jnp.int32)]
    + [pltpu.VMEM((_PC,), jnp.int32) for _ in range(6)]
    + [pltpu.VMEM((_PC, _D), jnp.float32) for _ in range(3)]
    + [pltpu.SemaphoreType.DMA for _ in range(6)]
    + [pltpu.VMEM_SHARED((_NP, _D), jnp.float32)]
)


@functools.partial(
    pl.kernel,
    out_type=jax.ShapeDtypeStruct((_NC, _NP, _D), jnp.float32),
    mesh=_mesh,
    scratch_types=_msg_scratch,
    compiler_params=_sc_params,
)
def _msg_kernel(y_hbm, pidx_hbm, acc_out, *sc):
    pidx_v = sc[0]
    slots = tuple((sc[1 + r], sc[4 + r], sc[7 + r], sc[10 + r], sc[13 + r])
                  for r in range(3))
    acc_sp = sc[16]

    cid = lax.axis_index("c")
    sid = lax.axis_index("s")
    wid = sid * _NC + cid

    pltpu.sync_copy(pidx_hbm.at[wid], pidx_v)

    buf0 = slots[0][2]
    _fill_const(buf0, _CB, _D, 0.0)
    for t in range(_NB):
        pltpu.async_copy(buf0.at[pl.ds(0, _CB)],
                         acc_sp.at[pl.ds(sid * _RPT + t * _CB, _CB)], slots[0][3])
    for t in range(_NB):
        pltpu.make_async_copy(buf0.at[pl.ds(0, _CB)],
                              acc_sp.at[pl.ds(sid * _RPT, _CB)], slots[0][3]).wait()
    plsc.subcore_barrier()

    def unpack(v, sl):
        for c in range(_PC // 16):
            u = pidx_v[v, pl.ds(c * 16, 16)]
            sl[0][pl.ds(c * 16, 16)] = jnp.bitwise_and(u, 0xFFFF)
            sl[1][pl.ds(c * 16, 16)] = lax.shift_right_logical(u, 16)

    def gath(sl):
        pltpu.async_copy(y_hbm.at[sl[0]], sl[2], sl[3])

    def wait_g(sl):
        pltpu.make_async_copy(y_hbm.at[sl[0]], sl[2], sl[3]).wait()

    def scat(sl):
        pltpu.async_copy(sl[2], acc_sp.at[sl[1]], sl[4], add=True)

    def wait_s(sl):
        pltpu.make_async_copy(sl[2], acc_sp.at[sl[1]], sl[4]).wait()

    def step(i, c_sl, n_sl, first=False):
        nxt = slots[n_sl]
        if not first:
            wait_s(nxt)
        unpack(i + 1, nxt)
        gath(nxt)
        cur = slots[c_sl]
        wait_g(cur)
        scat(cur)

    unpack(0, slots[0])
    gath(slots[0])
    step(0, 0, 1, first=True)
    step(1, 1, 2, first=True)

    def body(k, carry):
        for r in range(3):
            step(2 + 3 * k + r, (2 + r) % 3, r)
        return carry

    lax.fori_loop(0, 40, body, 0)

    step(122, 2, 0)
    step(123, 0, 1)
    last = slots[124 % 3]
    wait_g(last)
    scat(last)
    for r in range(3):
        wait_s(slots[r])

    plsc.subcore_barrier()
    cbuf = [slots[0][2], slots[1][2]]
    crd = [slots[0][3], slots[1][3]]
    cwr = [slots[0][4], slots[1][4]]
    for t in range(_NB):
        p = t % 2
        blk = pl.ds(sid * _RPT + t * _CB, _CB)
        if t >= 2:
            pltpu.make_async_copy(cbuf[p].at[pl.ds(0, _CB)],
                                  acc_out.at[cid, pl.ds(sid * _RPT, _CB)],
                                  cwr[p]).wait()
        pltpu.async_copy(acc_sp.at[blk], cbuf[p].at[pl.ds(0, _CB)], crd[p])
        pltpu.make_async_copy(acc_sp.at[pl.ds(sid * _RPT, _CB)],
                              cbuf[p].at[pl.ds(0, _CB)], crd[p]).wait()
        pltpu.async_copy(cbuf[p].at[pl.ds(0, _CB)], acc_out.at[cid, blk], cwr[p])
    for p in range(2):
        pltpu.make_async_copy(cbuf[p].at[pl.ds(0, _CB)],
                              acc_out.at[cid, pl.ds(sid * _RPT, _CB)],
                              cwr[p]).wait()


def _mm_body(x_ref, w_ref, xw_ref):
    xw_ref[...] = jnp.dot(x_ref[...], w_ref[...], preferred_element_type=jnp.float32)


_mm = pl.pallas_call(
    _mm_body,
    out_shape=jax.ShapeDtypeStruct((_N, _D), jnp.float32),
)


def _scale_body(deg_ref, xw_ref, y_ref, dis_ref):
    deg = deg_ref[0][:_N, 0:1] + deg_ref[1][:_N, 0:1] + 1.0
    dis = lax.rsqrt(deg)
    y_ref[...] = xw_ref[...] * dis
    dis_ref[...] = dis


_scale = pl.pallas_call(
    _scale_body,
    out_shape=(
        jax.ShapeDtypeStruct((_N, _D), jnp.float32),
        jax.ShapeDtypeStruct((_N, 1), jnp.float32),
    ),
)


def _out_body(acc_ref, y_ref, dis_ref, b_ref, out_ref):
    out_ref[...] = (acc_ref[0][:_N] + acc_ref[1][:_N] + y_ref[...]) * dis_ref[...] + b_ref[...]


_outk = pl.pallas_call(
    _out_body,
    out_shape=jax.ShapeDtypeStruct((_N, _D), jnp.float32),
)


@jax.jit
def _run(x, edge_index, W, b):
    src = edge_index[0].reshape(_NW, _EPW)
    dst = edge_index[1].reshape(_NW, _EPW)
    dpad = _IR * _IC - _EPW
    dst_deg = jnp.pad(dst, ((0, 0), (0, dpad)), constant_values=_N).reshape(_NW, _IR, _IC)
    packed = jnp.bitwise_or(src, jnp.left_shift(dst, 16))
    ppad = _PR * _PC - _EPW
    packed = jnp.pad(packed, ((0, 0), (0, ppad)),
                     constant_values=_N << 16).reshape(_NW, _PR, _PC)
    degp = _deg_kernel(dst_deg)
    xw = _mm(x, W)
    y, dis = _scale(degp, xw)
    accp = _msg_kernel(y, packed)
    return _outk(accp, y, dis, b.reshape(1, _D))


def kernel(x, edge_index, W, b):
    return _run(x, edge_index, W, b)
